# baseline (speedup 1.0000x reference)
import functools

import jax
import jax.numpy as jnp
from jax import lax
from jax.experimental import pallas as pl
from jax.experimental.pallas import tpu as pltpu
from jax.experimental.pallas import tpu_sc as plsc

D_MODEL = 1024
D_FF = 4096
N_EXP = 8
T = 2048

R_BLK = 256
N_TILES = 24
N_CAP = N_TILES * R_BLK

NC = 2
NS = 16
LANES = 16

TOK_W = T // NS
ROW_W = N_CAP // (NC * NS)
TOK_GW = T // (NC * NS)

@functools.cache
def _mesh():
    return plsc.VectorSubcoreMesh(
        core_axis_name="c", subcore_axis_name="s",
        num_cores=NC, num_subcores=NS)


def _gate_body(x_ref, wg_ref, pos0_ref, pos1_ref, w0_ref, w1_ref, te_ref):
    x = x_ref[...]
    logits = lax.dot_general(x, wg_ref[...], (((1,), (0,)), ((), ())),
                             preferred_element_type=jnp.float32)
    g = jax.nn.softmax(logits, axis=-1)
    iota = lax.broadcasted_iota(jnp.int32, g.shape, 1)
    i1 = jnp.argmax(g, axis=-1)
    m1 = jnp.max(g, axis=-1, keepdims=True)
    gm = jnp.where(iota == i1[:, None], -1.0, g)
    i2 = jnp.argmax(gm, axis=-1)
    m2 = jnp.max(gm, axis=-1, keepdims=True)
    s = m1 + m2
    w0_ref[...] = m1 / s
    w1_ref[...] = m2 / s

    oh1 = (iota == i1[:, None]).astype(jnp.int32)
    oh2 = (iota == i2[:, None]).astype(jnp.int32)
    cum = oh1 + oh2
    sh = 1
    while sh < T:
        z = jnp.zeros((sh, N_EXP), jnp.int32)
        cum = cum + jnp.concatenate([z, cum[:T - sh]], axis=0)
        sh *= 2
    gc = cum[T - 1:T, :]
    padded = ((gc + (R_BLK - 1)) // R_BLK) * R_BLK
    tri = (lax.broadcasted_iota(jnp.int32, (N_EXP, N_EXP), 0)
           < lax.broadcasted_iota(jnp.int32, (N_EXP, N_EXP), 1)
           ).astype(jnp.float32)
    off = lax.dot_general(padded.astype(jnp.float32), tri,
                          (((1,), (0,)), ((), ())),
                          preferred_element_type=jnp.float32
                          ).astype(jnp.int32)
    dest = off + cum - 1
    pos0_ref[...] = jnp.sum(jnp.where(oh1 > 0, dest, 0), axis=1,
                            keepdims=True)
    pos1_ref[...] = jnp.sum(jnp.where(oh2 > 0, dest, 0), axis=1,
                            keepdims=True)

    tbase = lax.broadcasted_iota(jnp.int32, (32, N_EXP), 0) * R_BLK
    eidx = lax.broadcasted_iota(jnp.int32, (32, N_EXP), 1)
    offb = jnp.broadcast_to(off, (32, N_EXP))
    te_ref[...] = jnp.sum(
        jnp.where((eidx > 0) & (tbase >= offb), 1, 0), axis=1, keepdims=True)


def _gate(flat, Wg):
    return pl.pallas_call(
        _gate_body,
        in_specs=[pl.BlockSpec((T, D_MODEL), lambda: (0, 0)),
                  pl.BlockSpec((D_MODEL, N_EXP), lambda: (0, 0))],
        out_specs=[pl.BlockSpec((T, 1), lambda: (0, 0)),
                   pl.BlockSpec((T, 1), lambda: (0, 0)),
                   pl.BlockSpec((T, 1), lambda: (0, 0)),
                   pl.BlockSpec((T, 1), lambda: (0, 0)),
                   pl.BlockSpec((32, 1), lambda: (0, 0))],
        out_shape=[jax.ShapeDtypeStruct((T, 1), jnp.int32),
                   jax.ShapeDtypeStruct((T, 1), jnp.int32),
                   jax.ShapeDtypeStruct((T, 1), jnp.float32),
                   jax.ShapeDtypeStruct((T, 1), jnp.float32),
                   jax.ShapeDtypeStruct((32, 1), jnp.int32)],
    )(flat, Wg)


ROW_SH = N_CAP // NS


def _scat_body(pos0_hbm, pos1_hbm, rowtok_hbm,
               posidx_v, tok_v, zer_v, stg_v, rowtok_sh):
    s = lax.axis_index("s")
    c = lax.axis_index("c")
    lane = lax.iota(jnp.int32, 16)
    tbase = pl.multiple_of(s * TOK_W, TOK_W)

    pltpu.sync_copy(pos0_hbm.at[pl.ds(tbase, TOK_W)],
                    posidx_v.at[pl.ds(0, TOK_W)])
    pltpu.sync_copy(pos1_hbm.at[pl.ds(tbase, TOK_W)],
                    posidx_v.at[pl.ds(TOK_W, TOK_W)])
    for slot in range(2):
        for k in range(TOK_W // 16):
            tok_v[pl.ds(slot * TOK_W + k * 16, 16)] = (
                s * TOK_W + k * 16 + lane)
    for k in range(ROW_SH // 16):
        zer_v[pl.ds(k * 16, 16)] = jnp.zeros((16,), jnp.int32)
    sbase = pl.multiple_of(s * ROW_SH, ROW_SH)
    pltpu.sync_copy(zer_v, rowtok_sh.at[pl.ds(sbase, ROW_SH)])
    plsc.subcore_barrier()
    pltpu.sync_copy(tok_v, rowtok_sh.at[posidx_v], add=True)
    plsc.subcore_barrier()

    @pl.when(c == 0)
    def _write():
        pltpu.sync_copy(rowtok_sh.at[pl.ds(sbase, ROW_SH)], stg_v)
        pltpu.sync_copy(stg_v, rowtok_hbm.at[pl.ds(sbase, ROW_SH)])


@functools.cache
def _scat():
    return pl.kernel(
        _scat_body,
        out_type=jax.ShapeDtypeStruct((N_CAP,), jnp.int32),
        mesh=_mesh(),
        scratch_types=[
            pltpu.VMEM((2 * TOK_W,), jnp.int32),
            pltpu.VMEM((2 * TOK_W,), jnp.int32),
            pltpu.VMEM((ROW_SH,), jnp.int32),
            pltpu.VMEM((ROW_SH,), jnp.int32),
            pltpu.VMEM_SHARED((N_CAP,), jnp.int32),
        ])


G_CHUNK = 64


def _gatherx_body(xb_hbm, rowtok_hbm, xg_hbm, idx_v, rows_v, sem):
    s = lax.axis_index("s")
    c = lax.axis_index("c")
    wid = s * NC + c
    base = pl.multiple_of(wid * ROW_W, 8)
    for k in range(ROW_W // G_CHUNK):
        o = pl.multiple_of(base + k * G_CHUNK, 8)
        pltpu.sync_copy(rowtok_hbm.at[pl.ds(o, G_CHUNK)], idx_v)
        pltpu.async_copy(xb_hbm.at[idx_v], rows_v, sem).wait()
        pltpu.sync_copy(rows_v, xg_hbm.at[pl.ds(o, G_CHUNK)])


@functools.cache
def _gatherx():
    return pl.kernel(
        _gatherx_body,
        out_type=jax.ShapeDtypeStruct((N_CAP, D_MODEL // 2), jnp.int32),
        mesh=_mesh(),
        scratch_types=[
            pltpu.VMEM((G_CHUNK,), jnp.int32),
            pltpu.VMEM((G_CHUNK, D_MODEL // 2), jnp.int32),
            pltpu.SemaphoreType.DMA,
        ])


def _ffn_body(te_ref, xg_ref, w1_ref, b1_ref, w2_ref, b2_ref, og_ref):
    h = lax.dot_general(xg_ref[...], w1_ref[0], (((1,), (0,)), ((), ())),
                        preferred_element_type=jnp.float32) + b1_ref[0]
    h = jax.nn.gelu(h).astype(jnp.bfloat16)
    og_ref[...] = lax.dot_general(h, w2_ref[0], (((1,), (0,)), ((), ())),
                                  preferred_element_type=jnp.float32) + b2_ref[0]


def _ffn(te, xgb, W1b, b1r, W2b, b2r):
    grid_spec = pltpu.PrefetchScalarGridSpec(
        num_scalar_prefetch=1,
        grid=(N_TILES,),
        in_specs=[
            pl.BlockSpec((R_BLK, D_MODEL), lambda r, te: (r, 0)),
            pl.BlockSpec((1, D_MODEL, D_FF), lambda r, te: (te[r], 0, 0)),
            pl.BlockSpec((1, 1, D_FF), lambda r, te: (te[r], 0, 0)),
            pl.BlockSpec((1, D_FF, D_MODEL), lambda r, te: (te[r], 0, 0)),
            pl.BlockSpec((1, 1, D_MODEL), lambda r, te: (te[r], 0, 0)),
        ],
        out_specs=pl.BlockSpec((R_BLK, D_MODEL), lambda r, te: (r, 0)),
    )
    return pl.pallas_call(
        _ffn_body,
        grid_spec=grid_spec,
        out_shape=jax.ShapeDtypeStruct((N_CAP, D_MODEL), jnp.float32),
    )(te, xgb, W1b, b1r, W2b, b2r)


C_CHUNK = 32


def _gathero_body(og_hbm, pos0_hbm, pos1_hbm, sel0_hbm, sel1_hbm,
                  idx_v, rows_v, sem):
    s = lax.axis_index("s")
    c = lax.axis_index("c")
    wid = s * NC + c
    base = pl.multiple_of(wid * TOK_GW, 8)
    for pos_hbm, sel_hbm in ((pos0_hbm, sel0_hbm), (pos1_hbm, sel1_hbm)):
        for k in range(TOK_GW // C_CHUNK):
            o = pl.multiple_of(base + k * C_CHUNK, 8)
            pltpu.sync_copy(pos_hbm.at[pl.ds(o, C_CHUNK)], idx_v)
            pltpu.async_copy(og_hbm.at[idx_v], rows_v, sem).wait()
            pltpu.sync_copy(rows_v, sel_hbm.at[pl.ds(o, C_CHUNK)])


@functools.cache
def _gathero():
    return pl.kernel(
        _gathero_body,
        out_type=(jax.ShapeDtypeStruct((T, D_MODEL), jnp.float32),
                  jax.ShapeDtypeStruct((T, D_MODEL), jnp.float32)),
        mesh=_mesh(),
        scratch_types=[
            pltpu.VMEM((C_CHUNK,), jnp.int32),
            pltpu.VMEM((C_CHUNK, D_MODEL), jnp.float32),
            pltpu.SemaphoreType.DMA,
        ])


def _comb_body(s0_ref, s1_ref, w0_ref, w1_ref, out_ref):
    out_ref[...] = w0_ref[...] * s0_ref[...] + w1_ref[...] * s1_ref[...]


def _combine(s0, s1, w0, w1):
    blk = 256
    return pl.pallas_call(
        _comb_body,
        grid=(T // blk,),
        in_specs=[pl.BlockSpec((blk, D_MODEL), lambda i: (i, 0)),
                  pl.BlockSpec((blk, D_MODEL), lambda i: (i, 0)),
                  pl.BlockSpec((blk, 1), lambda i: (i, 0)),
                  pl.BlockSpec((blk, 1), lambda i: (i, 0))],
        out_specs=pl.BlockSpec((blk, D_MODEL), lambda i: (i, 0)),
        out_shape=jax.ShapeDtypeStruct((T, D_MODEL), jnp.float32),
    )(s0, s1, w0, w1)


def kernel(inputs, Wg, W1, b1, W2, b2):
    flat = inputs.reshape((-1, inputs.shape[-1]))
    flat_b = flat.astype(jnp.bfloat16)
    xb_i32 = lax.bitcast_convert_type(
        flat_b.reshape(T, D_MODEL // 2, 2), jnp.int32)

    pos0, pos1, w0, w1, te = _gate(flat, Wg)
    pos0 = pos0.reshape(T)
    pos1 = pos1.reshape(T)
    rowtok = _scat()(pos0, pos1)
    xg_i32 = _gatherx()(xb_i32, rowtok)
    xgb = lax.bitcast_convert_type(xg_i32, jnp.bfloat16).reshape(N_CAP, D_MODEL)

    og = _ffn(te.reshape(32), xgb,
              W1.astype(jnp.bfloat16), b1.reshape(N_EXP, 1, D_FF),
              W2.astype(jnp.bfloat16), b2.reshape(N_EXP, 1, D_MODEL))

    s0, s1 = _gathero()(og, pos0, pos1)
    out = _combine(s0, s1, w0, w1)
    return out.reshape(inputs.shape)


# speedup vs baseline: 1.3499x; 1.3499x over previous
import functools

import jax
import jax.numpy as jnp
from jax import lax
from jax.experimental import pallas as pl
from jax.experimental.pallas import tpu as pltpu
from jax.experimental.pallas import tpu_sc as plsc

D_MODEL = 1024
D_FF = 4096
N_EXP = 8
T = 2048

R_BLK = 256
N_TILES = 24
N_CAP = N_TILES * R_BLK

NC = 2
NS = 16

TOK_W = T // NS
ROW_W = N_CAP // (NC * NS)
TOK_GW = T // (NC * NS)


@functools.cache
def _mesh():
    return plsc.VectorSubcoreMesh(
        core_axis_name="c", subcore_axis_name="s",
        num_cores=NC, num_subcores=NS)


def _gate_body(x_ref, wg_ref, pos0_ref, pos1_ref, w0_ref, w1_ref, te_ref):
    x = x_ref[...]
    logits = lax.dot_general(x, wg_ref[...], (((1,), (0,)), ((), ())),
                             preferred_element_type=jnp.float32)
    g = jax.nn.softmax(logits, axis=-1)
    iota = lax.broadcasted_iota(jnp.int32, g.shape, 1)
    i1 = jnp.argmax(g, axis=-1)
    m1 = jnp.max(g, axis=-1, keepdims=True)
    gm = jnp.where(iota == i1[:, None], -1.0, g)
    i2 = jnp.argmax(gm, axis=-1)
    m2 = jnp.max(gm, axis=-1, keepdims=True)
    s = m1 + m2
    w0_ref[...] = m1 / s
    w1_ref[...] = m2 / s

    oh1 = (iota == i1[:, None]).astype(jnp.int32)
    oh2 = (iota == i2[:, None]).astype(jnp.int32)
    cum = oh1 + oh2
    sh = 1
    while sh < T:
        z = jnp.zeros((sh, N_EXP), jnp.int32)
        cum = cum + jnp.concatenate([z, cum[:T - sh]], axis=0)
        sh *= 2
    gc = cum[T - 1:T, :]
    padded = ((gc + (R_BLK - 1)) // R_BLK) * R_BLK
    tri = (lax.broadcasted_iota(jnp.int32, (N_EXP, N_EXP), 0)
           < lax.broadcasted_iota(jnp.int32, (N_EXP, N_EXP), 1)
           ).astype(jnp.float32)
    off = lax.dot_general(padded.astype(jnp.float32), tri,
                          (((1,), (0,)), ((), ())),
                          preferred_element_type=jnp.float32
                          ).astype(jnp.int32)
    dest = off + cum - 1
    pos0_ref[...] = jnp.sum(jnp.where(oh1 > 0, dest, 0), axis=1,
                            keepdims=True)
    pos1_ref[...] = jnp.sum(jnp.where(oh2 > 0, dest, 0), axis=1,
                            keepdims=True)

    tbase = lax.broadcasted_iota(jnp.int32, (32, N_EXP), 0) * R_BLK
    eidx = lax.broadcasted_iota(jnp.int32, (32, N_EXP), 1)
    offb = jnp.broadcast_to(off, (32, N_EXP))
    te_ref[...] = jnp.sum(
        jnp.where((eidx > 0) & (tbase >= offb), 1, 0), axis=1, keepdims=True)


def _gate(flat, Wg):
    return pl.pallas_call(
        _gate_body,
        in_specs=[pl.BlockSpec((T, D_MODEL), lambda: (0, 0)),
                  pl.BlockSpec((D_MODEL, N_EXP), lambda: (0, 0))],
        out_specs=[pl.BlockSpec((T, 1), lambda: (0, 0)),
                   pl.BlockSpec((T, 1), lambda: (0, 0)),
                   pl.BlockSpec((T, 1), lambda: (0, 0)),
                   pl.BlockSpec((T, 1), lambda: (0, 0)),
                   pl.BlockSpec((32, 1), lambda: (0, 0))],
        out_shape=[jax.ShapeDtypeStruct((T, 1), jnp.int32),
                   jax.ShapeDtypeStruct((T, 1), jnp.int32),
                   jax.ShapeDtypeStruct((T, 1), jnp.float32),
                   jax.ShapeDtypeStruct((T, 1), jnp.float32),
                   jax.ShapeDtypeStruct((32, 1), jnp.int32)],
    )(flat, Wg)


ROW_SH = N_CAP // NS


def _scat_body(pos0_hbm, pos1_hbm, rowtok_hbm,
               idx0_v, idx1_v, tok0_v, tok1_v, zer_v, stg_v, rowtok_sh):
    s = lax.axis_index("s")
    c = lax.axis_index("c")
    lane = lax.iota(jnp.int32, 16)
    tbase = pl.multiple_of(s * TOK_W, TOK_W)

    pltpu.sync_copy(pos0_hbm.at[pl.ds(tbase, TOK_W)], idx0_v)
    pltpu.sync_copy(pos1_hbm.at[pl.ds(tbase, TOK_W)], idx1_v)
    for k in range(TOK_W // 16):
        tok = s * TOK_W + k * 16 + lane
        tok0_v[pl.ds(k * 16, 16)] = tok
        tok1_v[pl.ds(k * 16, 16)] = tok
    for k in range(ROW_SH // 16):
        zer_v[pl.ds(k * 16, 16)] = jnp.zeros((16,), jnp.int32)
    sbase = pl.multiple_of(s * ROW_SH, ROW_SH)
    pltpu.sync_copy(zer_v, rowtok_sh.at[pl.ds(sbase, ROW_SH)])
    plsc.subcore_barrier()
    pltpu.sync_copy(tok0_v, rowtok_sh.at[idx0_v], add=True)
    pltpu.sync_copy(tok1_v, rowtok_sh.at[idx1_v], add=True)
    plsc.subcore_barrier()

    @pl.when(c == 0)
    def _write():
        pltpu.sync_copy(rowtok_sh.at[pl.ds(sbase, ROW_SH)], stg_v)
        pltpu.sync_copy(stg_v, rowtok_hbm.at[pl.ds(sbase, ROW_SH)])


@functools.cache
def _scat():
    return pl.kernel(
        _scat_body,
        out_type=jax.ShapeDtypeStruct((N_CAP,), jnp.int32),
        mesh=_mesh(),
        scratch_types=[
            pltpu.VMEM((TOK_W,), jnp.int32),
            pltpu.VMEM((TOK_W,), jnp.int32),
            pltpu.VMEM((TOK_W,), jnp.int32),
            pltpu.VMEM((TOK_W,), jnp.int32),
            pltpu.VMEM((ROW_SH,), jnp.int32),
            pltpu.VMEM((ROW_SH,), jnp.int32),
            pltpu.VMEM_SHARED((N_CAP,), jnp.int32),
        ])


def _pipe(n, nb, get_fn, put_fn):
    gets = [None] * n
    puts = [None] * n
    for j in range(n):
        if j >= nb:
            puts[j - nb].wait()
        gets[j] = get_fn(j)
        if j >= 1:
            gets[j - 1].wait()
            puts[j - 1] = put_fn(j - 1)
    gets[n - 1].wait()
    puts[n - 1] = put_fn(n - 1)
    for j in range(max(0, n - nb), n):
        puts[j].wait()


GX_CHUNK = 24
GX_NB = 4
GX_N = ROW_W // GX_CHUNK


def _gatherx_body(x_hbm, rowtok_hbm, xg_hbm, idx_v, bufs, gsem, wsem):
    s = lax.axis_index("s")
    c = lax.axis_index("c")
    wid = s * NC + c
    base = pl.multiple_of(wid * ROW_W, 8)
    pltpu.sync_copy(rowtok_hbm.at[pl.ds(base, ROW_W)], idx_v)

    def get(j):
        return pltpu.async_copy(
            x_hbm.at[idx_v.at[pl.ds(j * GX_CHUNK, GX_CHUNK)]],
            bufs[j % GX_NB], gsem)

    def put(j):
        o = pl.multiple_of(base + j * GX_CHUNK, 8)
        return pltpu.async_copy(
            bufs[j % GX_NB], xg_hbm.at[pl.ds(o, GX_CHUNK)], wsem)

    _pipe(GX_N, GX_NB, get, put)


@functools.cache
def _gatherx():
    return pl.kernel(
        _gatherx_body,
        out_type=jax.ShapeDtypeStruct((N_CAP, D_MODEL), jnp.float32),
        mesh=_mesh(),
        scratch_types=[
            pltpu.VMEM((ROW_W,), jnp.int32),
            [pltpu.VMEM((GX_CHUNK, D_MODEL), jnp.float32)
             for _ in range(GX_NB)],
            pltpu.SemaphoreType.DMA,
            pltpu.SemaphoreType.DMA,
        ])


def _ffn_body(te_ref, xg_ref, w1_ref, b1_ref, w2_ref, b2_ref, og_ref):
    xgb = xg_ref[...].astype(jnp.bfloat16)
    h = lax.dot_general(xgb, w1_ref[0], (((1,), (0,)), ((), ())),
                        preferred_element_type=jnp.float32) + b1_ref[0]
    h = jax.nn.gelu(h).astype(jnp.bfloat16)
    og_ref[...] = lax.dot_general(h, w2_ref[0], (((1,), (0,)), ((), ())),
                                  preferred_element_type=jnp.float32
                                  ) + b2_ref[0]


def _ffn(te, xg, W1b, b1r, W2b, b2r):
    grid_spec = pltpu.PrefetchScalarGridSpec(
        num_scalar_prefetch=1,
        grid=(N_TILES,),
        in_specs=[
            pl.BlockSpec((R_BLK, D_MODEL), lambda r, te: (r, 0)),
            pl.BlockSpec((1, D_MODEL, D_FF), lambda r, te: (te[r], 0, 0)),
            pl.BlockSpec((1, 1, D_FF), lambda r, te: (te[r], 0, 0)),
            pl.BlockSpec((1, D_FF, D_MODEL), lambda r, te: (te[r], 0, 0)),
            pl.BlockSpec((1, 1, D_MODEL), lambda r, te: (te[r], 0, 0)),
        ],
        out_specs=pl.BlockSpec((R_BLK, D_MODEL), lambda r, te: (r, 0)),
    )
    return pl.pallas_call(
        _ffn_body,
        grid_spec=grid_spec,
        out_shape=jax.ShapeDtypeStruct((N_CAP, D_MODEL), jnp.float32),
    )(te, xg, W1b, b1r, W2b, b2r)


GO_CHUNK = 16
GO_NB = 4
GO_N = 2 * TOK_GW // GO_CHUNK


def _gathero_body(og_hbm, pos0_hbm, pos1_hbm, sel0_hbm, sel1_hbm,
                  idx_v, bufs, gsem, wsem):
    s = lax.axis_index("s")
    c = lax.axis_index("c")
    wid = s * NC + c
    base = pl.multiple_of(wid * TOK_GW, 8)
    pltpu.sync_copy(pos0_hbm.at[pl.ds(base, TOK_GW)],
                    idx_v.at[pl.ds(0, TOK_GW)])
    pltpu.sync_copy(pos1_hbm.at[pl.ds(base, TOK_GW)],
                    idx_v.at[pl.ds(TOK_GW, TOK_GW)])
    half = GO_N // 2

    def get(j):
        return pltpu.async_copy(
            og_hbm.at[idx_v.at[pl.ds(j * GO_CHUNK, GO_CHUNK)]],
            bufs[j % GO_NB], gsem)

    def put(j):
        dst = sel0_hbm if j < half else sel1_hbm
        o = pl.multiple_of(base + (j % half) * GO_CHUNK, 8)
        return pltpu.async_copy(
            bufs[j % GO_NB], dst.at[pl.ds(o, GO_CHUNK)], wsem)

    _pipe(GO_N, GO_NB, get, put)


@functools.cache
def _gathero():
    return pl.kernel(
        _gathero_body,
        out_type=(jax.ShapeDtypeStruct((T, D_MODEL), jnp.float32),
                  jax.ShapeDtypeStruct((T, D_MODEL), jnp.float32)),
        mesh=_mesh(),
        scratch_types=[
            pltpu.VMEM((2 * TOK_GW,), jnp.int32),
            [pltpu.VMEM((GO_CHUNK, D_MODEL), jnp.float32)
             for _ in range(GO_NB)],
            pltpu.SemaphoreType.DMA,
            pltpu.SemaphoreType.DMA,
        ])


def _comb_body(s0_ref, s1_ref, w0_ref, w1_ref, out_ref):
    out_ref[...] = (w0_ref[...] * s0_ref[...] + w1_ref[...] * s1_ref[...])


def _combine(s0, s1, w0, w1):
    blk = 256
    return pl.pallas_call(
        _comb_body,
        grid=(T // blk,),
        in_specs=[pl.BlockSpec((blk, D_MODEL), lambda i: (i, 0)),
                  pl.BlockSpec((blk, D_MODEL), lambda i: (i, 0)),
                  pl.BlockSpec((blk, 1), lambda i: (i, 0)),
                  pl.BlockSpec((blk, 1), lambda i: (i, 0))],
        out_specs=pl.BlockSpec((blk, D_MODEL), lambda i: (i, 0)),
        out_shape=jax.ShapeDtypeStruct((T, D_MODEL), jnp.float32),
    )(s0, s1, w0, w1)


def kernel(inputs, Wg, W1, b1, W2, b2):
    flat = inputs.reshape((-1, inputs.shape[-1]))

    pos0, pos1, w0, w1, te = _gate(flat, Wg)
    pos0 = pos0.reshape(T)
    pos1 = pos1.reshape(T)
    rowtok = _scat()(pos0, pos1)
    xg = _gatherx()(flat, rowtok)

    og = _ffn(te.reshape(32), xg,
              W1.astype(jnp.bfloat16), b1.reshape(N_EXP, 1, D_FF),
              W2.astype(jnp.bfloat16), b2.reshape(N_EXP, 1, D_MODEL))

    s0, s1 = _gathero()(og, pos0, pos1)
    out = _combine(s0, s1, w0, w1)
    return out.reshape(inputs.shape)


# speedup vs baseline: 1.7654x; 1.3078x over previous
import functools

import jax
import jax.numpy as jnp
from jax import lax
from jax.experimental import pallas as pl
from jax.experimental.pallas import tpu as pltpu
from jax.experimental.pallas import tpu_sc as plsc

D_MODEL = 1024
D_FF = 4096
N_EXP = 8
T = 2048

R_BLK = 256
N_TILES = 24
N_CAP = N_TILES * R_BLK

NC = 2
NS = 16

TOK_W = T // NS
ROW_W = N_CAP // (NC * NS)
TOK_GW = T // (NC * NS)


@functools.cache
def _mesh():
    return plsc.VectorSubcoreMesh(
        core_axis_name="c", subcore_axis_name="s",
        num_cores=NC, num_subcores=NS)


def _gate_body(x_ref, wg_ref, pos0_ref, pos1_ref, w0_ref, w1_ref, te_ref):
    x = x_ref[...]
    logits = lax.dot_general(x, wg_ref[...], (((1,), (0,)), ((), ())),
                             preferred_element_type=jnp.float32)
    g = jax.nn.softmax(logits, axis=-1)
    iota = lax.broadcasted_iota(jnp.int32, g.shape, 1)
    i1 = jnp.argmax(g, axis=-1)
    m1 = jnp.max(g, axis=-1, keepdims=True)
    gm = jnp.where(iota == i1[:, None], -1.0, g)
    i2 = jnp.argmax(gm, axis=-1)
    m2 = jnp.max(gm, axis=-1, keepdims=True)
    s = m1 + m2
    w0_ref[...] = m1 / s
    w1_ref[...] = m2 / s

    oh1 = (iota == i1[:, None]).astype(jnp.int32)
    oh2 = (iota == i2[:, None]).astype(jnp.int32)
    cum = oh1 + oh2
    sh = 1
    while sh < T:
        z = jnp.zeros((sh, N_EXP), jnp.int32)
        cum = cum + jnp.concatenate([z, cum[:T - sh]], axis=0)
        sh *= 2
    gc = cum[T - 1:T, :]
    padded = ((gc + (R_BLK - 1)) // R_BLK) * R_BLK
    tri = (lax.broadcasted_iota(jnp.int32, (N_EXP, N_EXP), 0)
           < lax.broadcasted_iota(jnp.int32, (N_EXP, N_EXP), 1)
           ).astype(jnp.float32)
    off = lax.dot_general(padded.astype(jnp.float32), tri,
                          (((1,), (0,)), ((), ())),
                          preferred_element_type=jnp.float32
                          ).astype(jnp.int32)
    dest = off + cum - 1
    pos0_ref[...] = jnp.sum(jnp.where(oh1 > 0, dest, 0), axis=1,
                            keepdims=True)
    pos1_ref[...] = jnp.sum(jnp.where(oh2 > 0, dest, 0), axis=1,
                            keepdims=True)

    tbase = lax.broadcasted_iota(jnp.int32, (32, N_EXP), 0) * R_BLK
    eidx = lax.broadcasted_iota(jnp.int32, (32, N_EXP), 1)
    offb = jnp.broadcast_to(off, (32, N_EXP))
    te_ref[...] = jnp.sum(
        jnp.where((eidx > 0) & (tbase >= offb), 1, 0), axis=1, keepdims=True)


def _gate(flat, Wg):
    return pl.pallas_call(
        _gate_body,
        in_specs=[pl.BlockSpec((T, D_MODEL), lambda: (0, 0)),
                  pl.BlockSpec((D_MODEL, N_EXP), lambda: (0, 0))],
        out_specs=[pl.BlockSpec((T, 1), lambda: (0, 0)),
                   pl.BlockSpec((T, 1), lambda: (0, 0)),
                   pl.BlockSpec((T, 1), lambda: (0, 0)),
                   pl.BlockSpec((T, 1), lambda: (0, 0)),
                   pl.BlockSpec((32, 1), lambda: (0, 0))],
        out_shape=[jax.ShapeDtypeStruct((T, 1), jnp.int32),
                   jax.ShapeDtypeStruct((T, 1), jnp.int32),
                   jax.ShapeDtypeStruct((T, 1), jnp.float32),
                   jax.ShapeDtypeStruct((T, 1), jnp.float32),
                   jax.ShapeDtypeStruct((32, 1), jnp.int32)],
    )(flat, Wg)


ROW_SH = N_CAP // NS


def _scat_body(pos0_hbm, pos1_hbm, rowtok_hbm,
               idx0_v, idx1_v, tok0_v, tok1_v, zer_v, stg_v, rowtok_sh):
    s = lax.axis_index("s")
    c = lax.axis_index("c")
    lane = lax.iota(jnp.int32, 16)
    tbase = pl.multiple_of(s * TOK_W, TOK_W)

    pltpu.sync_copy(pos0_hbm.at[pl.ds(tbase, TOK_W)], idx0_v)
    pltpu.sync_copy(pos1_hbm.at[pl.ds(tbase, TOK_W)], idx1_v)
    for k in range(TOK_W // 16):
        tok = s * TOK_W + k * 16 + lane
        tok0_v[pl.ds(k * 16, 16)] = tok
        tok1_v[pl.ds(k * 16, 16)] = tok
    sbase = pl.multiple_of(s * ROW_SH, ROW_SH)
    for k in range(ROW_SH // 16):
        zer_v[pl.ds(k * 16, 16)] = jnp.bitwise_and(
            sbase + k * 16 + lane, T - 1)
    pltpu.sync_copy(zer_v, rowtok_sh.at[pl.ds(sbase, ROW_SH)])
    plsc.subcore_barrier()
    pltpu.sync_copy(tok0_v, rowtok_sh.at[idx0_v])
    pltpu.sync_copy(tok1_v, rowtok_sh.at[idx1_v])
    plsc.subcore_barrier()

    @pl.when(c == 0)
    def _write():
        pltpu.sync_copy(rowtok_sh.at[pl.ds(sbase, ROW_SH)], stg_v)
        pltpu.sync_copy(stg_v, rowtok_hbm.at[pl.ds(sbase, ROW_SH)])


@functools.cache
def _scat():
    return pl.kernel(
        _scat_body,
        out_type=jax.ShapeDtypeStruct((N_CAP,), jnp.int32),
        mesh=_mesh(),
        scratch_types=[
            pltpu.VMEM((TOK_W,), jnp.int32),
            pltpu.VMEM((TOK_W,), jnp.int32),
            pltpu.VMEM((TOK_W,), jnp.int32),
            pltpu.VMEM((TOK_W,), jnp.int32),
            pltpu.VMEM((ROW_SH,), jnp.int32),
            pltpu.VMEM((ROW_SH,), jnp.int32),
            pltpu.VMEM_SHARED((N_CAP,), jnp.int32),
        ])


def _pipe(n, nb, get_fn, put_fn):
    gets = [None] * n
    puts = [None] * n
    for j in range(n):
        if j >= nb:
            puts[j - nb].wait()
        gets[j] = get_fn(j)
        if j >= 1:
            gets[j - 1].wait()
            puts[j - 1] = put_fn(j - 1)
    gets[n - 1].wait()
    puts[n - 1] = put_fn(n - 1)
    for j in range(max(0, n - nb), n):
        puts[j].wait()


GX_CHUNK = 24
GX_NB = 4
GX_N = ROW_W // GX_CHUNK


def _gatherx_body(x_hbm, rowtok_hbm, xg_hbm, idx_v, bufs, gsem, wsem):
    s = lax.axis_index("s")
    c = lax.axis_index("c")
    wid = s * NC + c
    base = pl.multiple_of(wid * ROW_W, 8)
    pltpu.sync_copy(rowtok_hbm.at[pl.ds(base, ROW_W)], idx_v)

    def get(j):
        return pltpu.async_copy(
            x_hbm.at[idx_v.at[pl.ds(j * GX_CHUNK, GX_CHUNK)]],
            bufs[j % GX_NB], gsem)

    def put(j):
        o = pl.multiple_of(base + j * GX_CHUNK, 8)
        return pltpu.async_copy(
            bufs[j % GX_NB], xg_hbm.at[pl.ds(o, GX_CHUNK)], wsem)

    _pipe(GX_N, GX_NB, get, put)


@functools.cache
def _gatherx():
    return pl.kernel(
        _gatherx_body,
        out_type=jax.ShapeDtypeStruct((N_CAP, D_MODEL), jnp.float32),
        mesh=_mesh(),
        scratch_types=[
            pltpu.VMEM((ROW_W,), jnp.int32),
            [pltpu.VMEM((GX_CHUNK, D_MODEL), jnp.float32)
             for _ in range(GX_NB)],
            pltpu.SemaphoreType.DMA,
            pltpu.SemaphoreType.DMA,
        ])


def _ffn_body(te_ref, xg_ref, w1_ref, b1_ref, w2_ref, b2_ref, og_ref):
    xgb = xg_ref[...].astype(jnp.bfloat16)
    h = lax.dot_general(xgb, w1_ref[0], (((1,), (0,)), ((), ())),
                        preferred_element_type=jnp.float32) + b1_ref[0]
    h = jax.nn.gelu(h).astype(jnp.bfloat16)
    og_ref[...] = lax.dot_general(h, w2_ref[0], (((1,), (0,)), ((), ())),
                                  preferred_element_type=jnp.float32
                                  ) + b2_ref[0]


def _ffn(te, xg, W1b, b1r, W2b, b2r):
    grid_spec = pltpu.PrefetchScalarGridSpec(
        num_scalar_prefetch=1,
        grid=(N_TILES,),
        in_specs=[
            pl.BlockSpec((R_BLK, D_MODEL), lambda r, te: (r, 0)),
            pl.BlockSpec((1, D_MODEL, D_FF), lambda r, te: (te[r], 0, 0)),
            pl.BlockSpec((1, 1, D_FF), lambda r, te: (te[r], 0, 0)),
            pl.BlockSpec((1, D_FF, D_MODEL), lambda r, te: (te[r], 0, 0)),
            pl.BlockSpec((1, 1, D_MODEL), lambda r, te: (te[r], 0, 0)),
        ],
        out_specs=pl.BlockSpec((R_BLK, D_MODEL), lambda r, te: (r, 0)),
    )
    return pl.pallas_call(
        _ffn_body,
        grid_spec=grid_spec,
        out_shape=jax.ShapeDtypeStruct((N_CAP, D_MODEL), jnp.float32),
    )(te, xg, W1b, b1r, W2b, b2r)


GO_CHUNK = 16
GO_NB = 4
GO_N = 2 * TOK_GW // GO_CHUNK


def _gathero_body(og_hbm, pos0_hbm, pos1_hbm, sel0_hbm, sel1_hbm,
                  idx_v, bufs, gsem, wsem):
    s = lax.axis_index("s")
    c = lax.axis_index("c")
    wid = s * NC + c
    base = pl.multiple_of(wid * TOK_GW, 8)
    pltpu.sync_copy(pos0_hbm.at[pl.ds(base, TOK_GW)],
                    idx_v.at[pl.ds(0, TOK_GW)])
    pltpu.sync_copy(pos1_hbm.at[pl.ds(base, TOK_GW)],
                    idx_v.at[pl.ds(TOK_GW, TOK_GW)])
    half = GO_N // 2

    def get(j):
        return pltpu.async_copy(
            og_hbm.at[idx_v.at[pl.ds(j * GO_CHUNK, GO_CHUNK)]],
            bufs[j % GO_NB], gsem)

    def put(j):
        dst = sel0_hbm if j < half else sel1_hbm
        o = pl.multiple_of(base + (j % half) * GO_CHUNK, 8)
        return pltpu.async_copy(
            bufs[j % GO_NB], dst.at[pl.ds(o, GO_CHUNK)], wsem)

    _pipe(GO_N, GO_NB, get, put)


@functools.cache
def _gathero():
    return pl.kernel(
        _gathero_body,
        out_type=(jax.ShapeDtypeStruct((T, D_MODEL), jnp.float32),
                  jax.ShapeDtypeStruct((T, D_MODEL), jnp.float32)),
        mesh=_mesh(),
        scratch_types=[
            pltpu.VMEM((2 * TOK_GW,), jnp.int32),
            [pltpu.VMEM((GO_CHUNK, D_MODEL), jnp.float32)
             for _ in range(GO_NB)],
            pltpu.SemaphoreType.DMA,
            pltpu.SemaphoreType.DMA,
        ])


def _comb_body(s0_ref, s1_ref, w0_ref, w1_ref, out_ref):
    out_ref[...] = (w0_ref[...] * s0_ref[...] + w1_ref[...] * s1_ref[...])


def _combine(s0, s1, w0, w1):
    blk = 256
    return pl.pallas_call(
        _comb_body,
        grid=(T // blk,),
        in_specs=[pl.BlockSpec((blk, D_MODEL), lambda i: (i, 0)),
                  pl.BlockSpec((blk, D_MODEL), lambda i: (i, 0)),
                  pl.BlockSpec((blk, 1), lambda i: (i, 0)),
                  pl.BlockSpec((blk, 1), lambda i: (i, 0))],
        out_specs=pl.BlockSpec((blk, D_MODEL), lambda i: (i, 0)),
        out_shape=jax.ShapeDtypeStruct((T, D_MODEL), jnp.float32),
    )(s0, s1, w0, w1)


def kernel(inputs, Wg, W1, b1, W2, b2):
    flat = inputs.reshape((-1, inputs.shape[-1]))

    pos0, pos1, w0, w1, te = _gate(flat, Wg)
    pos0 = pos0.reshape(T)
    pos1 = pos1.reshape(T)
    rowtok = _scat()(pos0, pos1)
    xg = _gatherx()(flat, rowtok)

    og = _ffn(te.reshape(32), xg,
              W1.astype(jnp.bfloat16), b1.reshape(N_EXP, 1, D_FF),
              W2.astype(jnp.bfloat16), b2.reshape(N_EXP, 1, D_MODEL))

    s0, s1 = _gathero()(og, pos0, pos1)
    out = _combine(s0, s1, w0, w1)
    return out.reshape(inputs.shape)


# speedup vs baseline: 1.9938x; 1.1293x over previous
import functools

import jax
import jax.numpy as jnp
from jax import lax
from jax.experimental import pallas as pl
from jax.experimental.pallas import tpu as pltpu
from jax.experimental.pallas import tpu_sc as plsc

D_MODEL = 1024
D_FF = 4096
N_EXP = 8
T = 2048

R_BLK = 256
N_TILES = 24
N_CAP = N_TILES * R_BLK

NC = 2
NS = 16

TOK_W = T // NS
ROW_W = N_CAP // (NC * NS)
TOK_GW = T // (NC * NS)


@functools.cache
def _mesh():
    return plsc.VectorSubcoreMesh(
        core_axis_name="c", subcore_axis_name="s",
        num_cores=NC, num_subcores=NS)


def _gate_body(x_ref, wg_ref, pos0_ref, pos1_ref, w0_ref, w1_ref, te_ref):
    x = x_ref[...]
    logits = lax.dot_general(x, wg_ref[...], (((1,), (0,)), ((), ())),
                             preferred_element_type=jnp.float32)
    g = jax.nn.softmax(logits, axis=-1)
    iota = lax.broadcasted_iota(jnp.int32, g.shape, 1)
    i1 = jnp.argmax(g, axis=-1)
    m1 = jnp.max(g, axis=-1, keepdims=True)
    gm = jnp.where(iota == i1[:, None], -1.0, g)
    i2 = jnp.argmax(gm, axis=-1)
    m2 = jnp.max(gm, axis=-1, keepdims=True)
    s = m1 + m2
    w0_ref[...] = m1 / s
    w1_ref[...] = m2 / s

    oh1 = (iota == i1[:, None]).astype(jnp.int32)
    oh2 = (iota == i2[:, None]).astype(jnp.int32)
    cum = oh1 + oh2
    sh = 1
    while sh < T:
        z = jnp.zeros((sh, N_EXP), jnp.int32)
        cum = cum + jnp.concatenate([z, cum[:T - sh]], axis=0)
        sh *= 2
    gc = cum[T - 1:T, :]
    padded = ((gc + (R_BLK - 1)) // R_BLK) * R_BLK
    tri = (lax.broadcasted_iota(jnp.int32, (N_EXP, N_EXP), 0)
           < lax.broadcasted_iota(jnp.int32, (N_EXP, N_EXP), 1)
           ).astype(jnp.float32)
    off = lax.dot_general(padded.astype(jnp.float32), tri,
                          (((1,), (0,)), ((), ())),
                          preferred_element_type=jnp.float32
                          ).astype(jnp.int32)
    dest = off + cum - 1
    pos0_ref[...] = jnp.sum(jnp.where(oh1 > 0, dest, 0), axis=1,
                            keepdims=True)
    pos1_ref[...] = jnp.sum(jnp.where(oh2 > 0, dest, 0), axis=1,
                            keepdims=True)

    tbase = lax.broadcasted_iota(jnp.int32, (32, N_EXP), 0) * R_BLK
    eidx = lax.broadcasted_iota(jnp.int32, (32, N_EXP), 1)
    offb = jnp.broadcast_to(off, (32, N_EXP))
    te_ref[...] = jnp.sum(
        jnp.where((eidx > 0) & (tbase >= offb), 1, 0), axis=1, keepdims=True)


def _gate(flat, Wg):
    return pl.pallas_call(
        _gate_body,
        in_specs=[pl.BlockSpec((T, D_MODEL), lambda: (0, 0)),
                  pl.BlockSpec((D_MODEL, N_EXP), lambda: (0, 0))],
        out_specs=[pl.BlockSpec((T, 1), lambda: (0, 0)),
                   pl.BlockSpec((T, 1), lambda: (0, 0)),
                   pl.BlockSpec((T, 1), lambda: (0, 0)),
                   pl.BlockSpec((T, 1), lambda: (0, 0)),
                   pl.BlockSpec((32, 1), lambda: (0, 0))],
        out_shape=[jax.ShapeDtypeStruct((T, 1), jnp.int32),
                   jax.ShapeDtypeStruct((T, 1), jnp.int32),
                   jax.ShapeDtypeStruct((T, 1), jnp.float32),
                   jax.ShapeDtypeStruct((T, 1), jnp.float32),
                   jax.ShapeDtypeStruct((32, 1), jnp.int32)],
    )(flat, Wg)


ROW_SH = N_CAP // NS


def _scat_body(pos0_hbm, pos1_hbm, rowtok_hbm,
               idx0_v, idx1_v, tok0_v, tok1_v, zer_v, stg_v, rowtok_sh):
    s = lax.axis_index("s")
    c = lax.axis_index("c")
    lane = lax.iota(jnp.int32, 16)
    tbase = pl.multiple_of(s * TOK_W, TOK_W)

    pltpu.sync_copy(pos0_hbm.at[pl.ds(tbase, TOK_W)], idx0_v)
    pltpu.sync_copy(pos1_hbm.at[pl.ds(tbase, TOK_W)], idx1_v)
    for k in range(TOK_W // 16):
        tok = s * TOK_W + k * 16 + lane
        tok0_v[pl.ds(k * 16, 16)] = tok
        tok1_v[pl.ds(k * 16, 16)] = tok
    sbase = pl.multiple_of(s * ROW_SH, ROW_SH)
    for k in range(ROW_SH // 16):
        zer_v[pl.ds(k * 16, 16)] = jnp.bitwise_and(
            sbase + k * 16 + lane, T - 1)
    pltpu.sync_copy(zer_v, rowtok_sh.at[pl.ds(sbase, ROW_SH)])
    plsc.subcore_barrier()
    pltpu.sync_copy(tok0_v, rowtok_sh.at[idx0_v])
    pltpu.sync_copy(tok1_v, rowtok_sh.at[idx1_v])
    plsc.subcore_barrier()

    @pl.when(c == 0)
    def _write():
        pltpu.sync_copy(rowtok_sh.at[pl.ds(sbase, ROW_SH)], stg_v)
        pltpu.sync_copy(stg_v, rowtok_hbm.at[pl.ds(sbase, ROW_SH)])


@functools.cache
def _scat():
    return pl.kernel(
        _scat_body,
        out_type=jax.ShapeDtypeStruct((N_CAP,), jnp.int32),
        mesh=_mesh(),
        scratch_types=[
            pltpu.VMEM((TOK_W,), jnp.int32),
            pltpu.VMEM((TOK_W,), jnp.int32),
            pltpu.VMEM((TOK_W,), jnp.int32),
            pltpu.VMEM((TOK_W,), jnp.int32),
            pltpu.VMEM((ROW_SH,), jnp.int32),
            pltpu.VMEM((ROW_SH,), jnp.int32),
            pltpu.VMEM_SHARED((N_CAP,), jnp.int32),
        ])


def _pipe(n, nb, get_fn, put_fn):
    gets = [None] * n
    puts = [None] * n
    for j in range(n):
        if j >= nb:
            puts[j - nb].wait()
        gets[j] = get_fn(j)
        if j >= 1:
            gets[j - 1].wait()
            puts[j - 1] = put_fn(j - 1)
    gets[n - 1].wait()
    puts[n - 1] = put_fn(n - 1)
    for j in range(max(0, n - nb), n):
        puts[j].wait()


GX_CHUNK = 24
GX_NB = 4
GX_N = ROW_W // GX_CHUNK


def _gatherx_body(x_hbm, rowtok_hbm, xg_hbm, idx_v, bufs, gsem, wsem):
    s = lax.axis_index("s")
    c = lax.axis_index("c")
    wid = s * NC + c
    base = pl.multiple_of(wid * ROW_W, 8)
    pltpu.sync_copy(rowtok_hbm.at[pl.ds(base, ROW_W)], idx_v)

    def get(j):
        return pltpu.async_copy(
            x_hbm.at[idx_v.at[pl.ds(j * GX_CHUNK, GX_CHUNK)]],
            bufs[j % GX_NB], gsem)

    def put(j):
        o = pl.multiple_of(base + j * GX_CHUNK, 8)
        return pltpu.async_copy(
            bufs[j % GX_NB], xg_hbm.at[pl.ds(o, GX_CHUNK)], wsem)

    _pipe(GX_N, GX_NB, get, put)


@functools.cache
def _gatherx():
    return pl.kernel(
        _gatherx_body,
        out_type=jax.ShapeDtypeStruct((N_CAP, D_MODEL), jnp.float32),
        mesh=_mesh(),
        scratch_types=[
            pltpu.VMEM((ROW_W,), jnp.int32),
            [pltpu.VMEM((GX_CHUNK, D_MODEL), jnp.float32)
             for _ in range(GX_NB)],
            pltpu.SemaphoreType.DMA,
            pltpu.SemaphoreType.DMA,
        ])


def _ffn_body(te_ref, xg_ref, w1_ref, b1_ref, w2_ref, b2_ref, og_ref):
    h = lax.dot_general(xg_ref[...], w1_ref[0], (((1,), (0,)), ((), ())),
                        preferred_element_type=jnp.float32) + b1_ref[0]
    h = jax.nn.gelu(h).astype(jnp.bfloat16)
    og_ref[...] = lax.dot_general(h, w2_ref[0], (((1,), (0,)), ((), ())),
                                  preferred_element_type=jnp.float32
                                  ) + b2_ref[0]


def _ffn(te, xg, W1b, b1r, W2b, b2r):
    grid_spec = pltpu.PrefetchScalarGridSpec(
        num_scalar_prefetch=1,
        grid=(N_TILES,),
        in_specs=[
            pl.BlockSpec((R_BLK, D_MODEL), lambda r, te: (r, 0)),
            pl.BlockSpec((1, D_MODEL, D_FF), lambda r, te: (te[r], 0, 0)),
            pl.BlockSpec((1, 1, D_FF), lambda r, te: (te[r], 0, 0)),
            pl.BlockSpec((1, D_FF, D_MODEL), lambda r, te: (te[r], 0, 0)),
            pl.BlockSpec((1, 1, D_MODEL), lambda r, te: (te[r], 0, 0)),
        ],
        out_specs=pl.BlockSpec((R_BLK, D_MODEL), lambda r, te: (r, 0)),
    )
    return pl.pallas_call(
        _ffn_body,
        grid_spec=grid_spec,
        out_shape=jax.ShapeDtypeStruct((N_CAP, D_MODEL), jnp.float32),
    )(te, xg, W1b, b1r, W2b, b2r)


GO_CHUNK = 16
GO_NB = 4
GO_N = 2 * TOK_GW // GO_CHUNK


def _gathero_body(og_hbm, pos0_hbm, pos1_hbm, sel0_hbm, sel1_hbm,
                  idx_v, bufs, gsem, wsem):
    s = lax.axis_index("s")
    c = lax.axis_index("c")
    wid = s * NC + c
    base = pl.multiple_of(wid * TOK_GW, 8)
    pltpu.sync_copy(pos0_hbm.at[pl.ds(base, TOK_GW)],
                    idx_v.at[pl.ds(0, TOK_GW)])
    pltpu.sync_copy(pos1_hbm.at[pl.ds(base, TOK_GW)],
                    idx_v.at[pl.ds(TOK_GW, TOK_GW)])
    half = GO_N // 2

    def get(j):
        return pltpu.async_copy(
            og_hbm.at[idx_v.at[pl.ds(j * GO_CHUNK, GO_CHUNK)]],
            bufs[j % GO_NB], gsem)

    def put(j):
        dst = sel0_hbm if j < half else sel1_hbm
        o = pl.multiple_of(base + (j % half) * GO_CHUNK, 8)
        return pltpu.async_copy(
            bufs[j % GO_NB], dst.at[pl.ds(o, GO_CHUNK)], wsem)

    _pipe(GO_N, GO_NB, get, put)


@functools.cache
def _gathero():
    return pl.kernel(
        _gathero_body,
        out_type=(jax.ShapeDtypeStruct((T, D_MODEL), jnp.float32),
                  jax.ShapeDtypeStruct((T, D_MODEL), jnp.float32)),
        mesh=_mesh(),
        scratch_types=[
            pltpu.VMEM((2 * TOK_GW,), jnp.int32),
            [pltpu.VMEM((GO_CHUNK, D_MODEL), jnp.float32)
             for _ in range(GO_NB)],
            pltpu.SemaphoreType.DMA,
            pltpu.SemaphoreType.DMA,
        ])


def _comb_body(s0_ref, s1_ref, w0_ref, w1_ref, out_ref):
    out_ref[...] = (w0_ref[...] * s0_ref[...] + w1_ref[...] * s1_ref[...])


def _combine(s0, s1, w0, w1):
    blk = 256
    return pl.pallas_call(
        _comb_body,
        grid=(T // blk,),
        in_specs=[pl.BlockSpec((blk, D_MODEL), lambda i: (i, 0)),
                  pl.BlockSpec((blk, D_MODEL), lambda i: (i, 0)),
                  pl.BlockSpec((blk, 1), lambda i: (i, 0)),
                  pl.BlockSpec((blk, 1), lambda i: (i, 0))],
        out_specs=pl.BlockSpec((blk, D_MODEL), lambda i: (i, 0)),
        out_shape=jax.ShapeDtypeStruct((T, D_MODEL), jnp.float32),
    )(s0, s1, w0, w1)


def kernel(inputs, Wg, W1, b1, W2, b2):
    flat = inputs.reshape((-1, inputs.shape[-1]))

    pos0, pos1, w0, w1, te = _gate(flat, Wg)
    pos0 = pos0.reshape(T)
    pos1 = pos1.reshape(T)
    rowtok = _scat()(pos0, pos1)
    xg = _gatherx()(flat, rowtok)

    og = _ffn(te.reshape(32), xg,
              W1, b1.reshape(N_EXP, 1, D_FF),
              W2.astype(jnp.bfloat16), b2.reshape(N_EXP, 1, D_MODEL))

    s0, s1 = _gathero()(og, pos0, pos1)
    out = _combine(s0, s1, w0, w1)
    return out.reshape(inputs.shape)


# speedup vs baseline: 2.0038x; 1.0050x over previous
import functools

import jax
import jax.numpy as jnp
from jax import lax
from jax.experimental import pallas as pl
from jax.experimental.pallas import tpu as pltpu
from jax.experimental.pallas import tpu_sc as plsc

D_MODEL = 1024
D_FF = 4096
N_EXP = 8
T = 2048

R_BLK = 256
N_TILES = 24
N_CAP = N_TILES * R_BLK

NC = 2
NS = 16

TOK_W = T // NS
ROW_W = N_CAP // (NC * NS)
TOK_GW = T // (NC * NS)


@functools.cache
def _mesh():
    return plsc.VectorSubcoreMesh(
        core_axis_name="c", subcore_axis_name="s",
        num_cores=NC, num_subcores=NS)


def _gate_body(x_ref, wg_ref, pos0_ref, pos1_ref, w0_ref, w1_ref, te_ref):
    x = x_ref[...]
    logits = lax.dot_general(x, wg_ref[...], (((1,), (0,)), ((), ())),
                             preferred_element_type=jnp.float32)
    g = jax.nn.softmax(logits, axis=-1)
    iota = lax.broadcasted_iota(jnp.int32, g.shape, 1)
    i1 = jnp.argmax(g, axis=-1)
    m1 = jnp.max(g, axis=-1, keepdims=True)
    gm = jnp.where(iota == i1[:, None], -1.0, g)
    i2 = jnp.argmax(gm, axis=-1)
    m2 = jnp.max(gm, axis=-1, keepdims=True)
    s = m1 + m2
    w0_ref[...] = m1 / s
    w1_ref[...] = m2 / s

    oh1 = (iota == i1[:, None]).astype(jnp.int32)
    oh2 = (iota == i2[:, None]).astype(jnp.int32)
    cum = oh1 + oh2
    sh = 1
    while sh < T:
        z = jnp.zeros((sh, N_EXP), jnp.int32)
        cum = cum + jnp.concatenate([z, cum[:T - sh]], axis=0)
        sh *= 2
    gc = cum[T - 1:T, :]
    padded = ((gc + (R_BLK - 1)) // R_BLK) * R_BLK
    tri = (lax.broadcasted_iota(jnp.int32, (N_EXP, N_EXP), 0)
           < lax.broadcasted_iota(jnp.int32, (N_EXP, N_EXP), 1)
           ).astype(jnp.float32)
    off = lax.dot_general(padded.astype(jnp.float32), tri,
                          (((1,), (0,)), ((), ())),
                          preferred_element_type=jnp.float32
                          ).astype(jnp.int32)
    dest = off + cum - 1
    pos0_ref[...] = jnp.sum(jnp.where(oh1 > 0, dest, 0), axis=1,
                            keepdims=True)
    pos1_ref[...] = jnp.sum(jnp.where(oh2 > 0, dest, 0), axis=1,
                            keepdims=True)

    tbase = lax.broadcasted_iota(jnp.int32, (32, N_EXP), 0) * R_BLK
    eidx = lax.broadcasted_iota(jnp.int32, (32, N_EXP), 1)
    offb = jnp.broadcast_to(off, (32, N_EXP))
    te_ref[...] = jnp.sum(
        jnp.where((eidx > 0) & (tbase >= offb), 1, 0), axis=1, keepdims=True)


def _gate(flat, Wg):
    return pl.pallas_call(
        _gate_body,
        in_specs=[pl.BlockSpec((T, D_MODEL), lambda: (0, 0)),
                  pl.BlockSpec((D_MODEL, N_EXP), lambda: (0, 0))],
        out_specs=[pl.BlockSpec((T, 1), lambda: (0, 0)),
                   pl.BlockSpec((T, 1), lambda: (0, 0)),
                   pl.BlockSpec((T, 1), lambda: (0, 0)),
                   pl.BlockSpec((T, 1), lambda: (0, 0)),
                   pl.BlockSpec((32, 1), lambda: (0, 0))],
        out_shape=[jax.ShapeDtypeStruct((T, 1), jnp.int32),
                   jax.ShapeDtypeStruct((T, 1), jnp.int32),
                   jax.ShapeDtypeStruct((T, 1), jnp.float32),
                   jax.ShapeDtypeStruct((T, 1), jnp.float32),
                   jax.ShapeDtypeStruct((32, 1), jnp.int32)],
    )(flat, Wg)


ROW_SH = N_CAP // NS


def _scat_body(pos0_hbm, pos1_hbm, rowtok_hbm,
               idx0_v, idx1_v, tok0_v, tok1_v, zer_v, stg_v, rowtok_sh):
    s = lax.axis_index("s")
    c = lax.axis_index("c")
    lane = lax.iota(jnp.int32, 16)
    tbase = pl.multiple_of(s * TOK_W, TOK_W)

    pltpu.sync_copy(pos0_hbm.at[pl.ds(tbase, TOK_W)], idx0_v)
    pltpu.sync_copy(pos1_hbm.at[pl.ds(tbase, TOK_W)], idx1_v)
    for k in range(TOK_W // 16):
        tok = s * TOK_W + k * 16 + lane
        tok0_v[pl.ds(k * 16, 16)] = tok
        tok1_v[pl.ds(k * 16, 16)] = tok
    sbase = pl.multiple_of(s * ROW_SH, ROW_SH)
    for k in range(ROW_SH // 16):
        zer_v[pl.ds(k * 16, 16)] = jnp.bitwise_and(
            sbase + k * 16 + lane, T - 1)
    pltpu.sync_copy(zer_v, rowtok_sh.at[pl.ds(sbase, ROW_SH)])
    plsc.subcore_barrier()
    pltpu.sync_copy(tok0_v, rowtok_sh.at[idx0_v])
    pltpu.sync_copy(tok1_v, rowtok_sh.at[idx1_v])
    plsc.subcore_barrier()

    @pl.when(c == 0)
    def _write():
        pltpu.sync_copy(rowtok_sh.at[pl.ds(sbase, ROW_SH)], stg_v)
        pltpu.sync_copy(stg_v, rowtok_hbm.at[pl.ds(sbase, ROW_SH)])


@functools.cache
def _scat():
    return pl.kernel(
        _scat_body,
        out_type=jax.ShapeDtypeStruct((N_CAP,), jnp.int32),
        mesh=_mesh(),
        scratch_types=[
            pltpu.VMEM((TOK_W,), jnp.int32),
            pltpu.VMEM((TOK_W,), jnp.int32),
            pltpu.VMEM((TOK_W,), jnp.int32),
            pltpu.VMEM((TOK_W,), jnp.int32),
            pltpu.VMEM((ROW_SH,), jnp.int32),
            pltpu.VMEM((ROW_SH,), jnp.int32),
            pltpu.VMEM_SHARED((N_CAP,), jnp.int32),
        ])


def _pipe(n, nb, get_fn, put_fn):
    gets = [None] * n
    puts = [None] * n
    for j in range(n):
        if j >= nb:
            puts[j - nb].wait()
        gets[j] = get_fn(j)
        if j >= 1:
            gets[j - 1].wait()
            puts[j - 1] = put_fn(j - 1)
    gets[n - 1].wait()
    puts[n - 1] = put_fn(n - 1)
    for j in range(max(0, n - nb), n):
        puts[j].wait()


GX_CHUNK = 24
GX_NB = 4
GX_N = ROW_W // GX_CHUNK


def _gatherx_body(x_hbm, rowtok_hbm, xg_hbm, idx_v, bufs, gsem, wsem):
    s = lax.axis_index("s")
    c = lax.axis_index("c")
    wid = s * NC + c
    base = pl.multiple_of(wid * ROW_W, 8)
    pltpu.sync_copy(rowtok_hbm.at[pl.ds(base, ROW_W)], idx_v)

    def get(j):
        return pltpu.async_copy(
            x_hbm.at[idx_v.at[pl.ds(j * GX_CHUNK, GX_CHUNK)]],
            bufs[j % GX_NB], gsem)

    def put(j):
        o = pl.multiple_of(base + j * GX_CHUNK, 8)
        return pltpu.async_copy(
            bufs[j % GX_NB], xg_hbm.at[pl.ds(o, GX_CHUNK)], wsem)

    _pipe(GX_N, GX_NB, get, put)


@functools.cache
def _gatherx():
    return pl.kernel(
        _gatherx_body,
        out_type=jax.ShapeDtypeStruct((N_CAP, D_MODEL), jnp.float32),
        mesh=_mesh(),
        scratch_types=[
            pltpu.VMEM((ROW_W,), jnp.int32),
            [pltpu.VMEM((GX_CHUNK, D_MODEL), jnp.float32)
             for _ in range(GX_NB)],
            pltpu.SemaphoreType.DMA,
            pltpu.SemaphoreType.DMA,
        ])


NSPLIT = 4
FS = D_FF // NSPLIT


def _ffn_body(te_ref, xg_ref, w1_ref, b1_ref, w2_ref, b2_ref, og_ref):
    xg = xg_ref[...]
    acc = jnp.broadcast_to(b2_ref[0], (R_BLK, D_MODEL))
    for k in range(NSPLIT):
        hk = lax.dot_general(
            xg, w1_ref[0, :, k * FS:(k + 1) * FS], (((1,), (0,)), ((), ())),
            preferred_element_type=jnp.float32) + b1_ref[0, :, k * FS:(k + 1) * FS]
        hk = jax.nn.gelu(hk).astype(jnp.bfloat16)
        acc = acc + lax.dot_general(
            hk, w2_ref[0, k * FS:(k + 1) * FS, :], (((1,), (0,)), ((), ())),
            preferred_element_type=jnp.float32)
    og_ref[...] = acc


def _ffn(te, xg, W1b, b1r, W2b, b2r):
    grid_spec = pltpu.PrefetchScalarGridSpec(
        num_scalar_prefetch=1,
        grid=(N_TILES,),
        in_specs=[
            pl.BlockSpec((R_BLK, D_MODEL), lambda r, te: (r, 0)),
            pl.BlockSpec((1, D_MODEL, D_FF), lambda r, te: (te[r], 0, 0)),
            pl.BlockSpec((1, 1, D_FF), lambda r, te: (te[r], 0, 0)),
            pl.BlockSpec((1, D_FF, D_MODEL), lambda r, te: (te[r], 0, 0)),
            pl.BlockSpec((1, 1, D_MODEL), lambda r, te: (te[r], 0, 0)),
        ],
        out_specs=pl.BlockSpec((R_BLK, D_MODEL), lambda r, te: (r, 0)),
    )
    return pl.pallas_call(
        _ffn_body,
        grid_spec=grid_spec,
        out_shape=jax.ShapeDtypeStruct((N_CAP, D_MODEL), jnp.float32),
    )(te, xg, W1b, b1r, W2b, b2r)


GO_CHUNK = 16
GO_NB = 4
GO_N = 2 * TOK_GW // GO_CHUNK


def _gathero_body(og_hbm, pos0_hbm, pos1_hbm, sel0_hbm, sel1_hbm,
                  idx_v, bufs, gsem, wsem):
    s = lax.axis_index("s")
    c = lax.axis_index("c")
    wid = s * NC + c
    base = pl.multiple_of(wid * TOK_GW, 8)
    pltpu.sync_copy(pos0_hbm.at[pl.ds(base, TOK_GW)],
                    idx_v.at[pl.ds(0, TOK_GW)])
    pltpu.sync_copy(pos1_hbm.at[pl.ds(base, TOK_GW)],
                    idx_v.at[pl.ds(TOK_GW, TOK_GW)])
    half = GO_N // 2

    def get(j):
        return pltpu.async_copy(
            og_hbm.at[idx_v.at[pl.ds(j * GO_CHUNK, GO_CHUNK)]],
            bufs[j % GO_NB], gsem)

    def put(j):
        dst = sel0_hbm if j < half else sel1_hbm
        o = pl.multiple_of(base + (j % half) * GO_CHUNK, 8)
        return pltpu.async_copy(
            bufs[j % GO_NB], dst.at[pl.ds(o, GO_CHUNK)], wsem)

    _pipe(GO_N, GO_NB, get, put)


@functools.cache
def _gathero():
    return pl.kernel(
        _gathero_body,
        out_type=(jax.ShapeDtypeStruct((T, D_MODEL), jnp.float32),
                  jax.ShapeDtypeStruct((T, D_MODEL), jnp.float32)),
        mesh=_mesh(),
        scratch_types=[
            pltpu.VMEM((2 * TOK_GW,), jnp.int32),
            [pltpu.VMEM((GO_CHUNK, D_MODEL), jnp.float32)
             for _ in range(GO_NB)],
            pltpu.SemaphoreType.DMA,
            pltpu.SemaphoreType.DMA,
        ])


def _comb_body(s0_ref, s1_ref, w0_ref, w1_ref, out_ref):
    out_ref[...] = (w0_ref[...] * s0_ref[...] + w1_ref[...] * s1_ref[...])


def _combine(s0, s1, w0, w1):
    blk = 256
    return pl.pallas_call(
        _comb_body,
        grid=(T // blk,),
        in_specs=[pl.BlockSpec((blk, D_MODEL), lambda i: (i, 0)),
                  pl.BlockSpec((blk, D_MODEL), lambda i: (i, 0)),
                  pl.BlockSpec((blk, 1), lambda i: (i, 0)),
                  pl.BlockSpec((blk, 1), lambda i: (i, 0))],
        out_specs=pl.BlockSpec((blk, D_MODEL), lambda i: (i, 0)),
        out_shape=jax.ShapeDtypeStruct((T, D_MODEL), jnp.float32),
    )(s0, s1, w0, w1)


def kernel(inputs, Wg, W1, b1, W2, b2):
    flat = inputs.reshape((-1, inputs.shape[-1]))

    pos0, pos1, w0, w1, te = _gate(flat, Wg)
    pos0 = pos0.reshape(T)
    pos1 = pos1.reshape(T)
    rowtok = _scat()(pos0, pos1)
    xg = _gatherx()(flat, rowtok)

    og = _ffn(te.reshape(32), xg,
              W1, b1.reshape(N_EXP, 1, D_FF),
              W2.astype(jnp.bfloat16), b2.reshape(N_EXP, 1, D_MODEL))

    s0, s1 = _gathero()(og, pos0, pos1)
    out = _combine(s0, s1, w0, w1)
    return out.reshape(inputs.shape)
